# baseline (speedup 1.0000x reference)
import functools

import jax
import jax.numpy as jnp
from jax import lax
from jax.experimental import pallas as pl
from jax.experimental.pallas import tpu as pltpu
from jax.experimental.pallas import tpu_sc as plsc

_N_FIELDS = 26
_VOCAB = 100000
_DPAD = 56
_NC = 2
_NS = 16
_NW = _NC * _NS
_L = 16
_RPD = 128
_CHUNK = 1024


@functools.cache
def _gather_call(b_total):
    b_per_w = b_total // _NW
    n_chunks = b_per_w // _CHUNK
    n_dma = _CHUNK // _RPD
    mesh = plsc.VectorSubcoreMesh(core_axis_name="c", subcore_axis_name="s")

    @functools.partial(
        pl.kernel,
        out_type=jax.ShapeDtypeStruct((b_total, _DPAD), jnp.float32),
        mesh=mesh,
        scratch_types=[
            pltpu.VMEM((_CHUNK,), jnp.int32),
            pltpu.VMEM((n_dma, _RPD), jnp.int32),
            pltpu.VMEM((_CHUNK, _DPAD), jnp.float32),
            pltpu.SemaphoreType.DMA,
        ],
        compiler_params=pltpu.CompilerParams(use_tc_tiling_on_sc=False),
    )
    def k(tab_hbm, idx_hbm, out_hbm, idx_raw, idx2d, rows, sem):
        wid = lax.axis_index("s") * _NC + lax.axis_index("c")
        base = wid * b_per_w
        iota = lax.iota(jnp.int32, _L)

        def chunk_body(c, carry):
            start = base + c * _CHUNK
            pltpu.sync_copy(idx_hbm.at[pl.ds(start, _CHUNK)], idx_raw)
            for i in range(_CHUNK // _L):
                pos = start + i * _L + iota
                f = lax.rem(pos, _N_FIELDS)
                g = idx_raw[pl.ds(i * _L, _L)] + f * _VOCAB
                idx2d[i // (_RPD // _L), pl.ds((i % (_RPD // _L)) * _L, _L)] = g
            descs = [
                pltpu.async_copy(
                    tab_hbm.at[idx2d.at[j]],
                    rows.at[pl.ds(j * _RPD, _RPD)],
                    sem,
                )
                for j in range(n_dma)
            ]
            for d in descs:
                d.wait()
            pltpu.sync_copy(rows, out_hbm.at[pl.ds(start, _CHUNK)])
            return carry

        lax.fori_loop(0, n_chunks, chunk_body, 0)

    return k


def kernel(cat, tables):
    batch, n_fields = cat.shape
    _, vocab, d = tables.shape
    tab = jnp.pad(tables.reshape(n_fields * vocab, d), ((0, 0), (0, _DPAD - d)))
    idx = cat.reshape(-1)
    out = _gather_call(batch * n_fields)(tab, idx)
    return out[:, :d].reshape(batch, n_fields * d)


# speedup vs baseline: 5.2718x; 5.2718x over previous
import functools

import jax
import jax.numpy as jnp
from jax import lax
from jax.experimental import pallas as pl
from jax.experimental.pallas import tpu as pltpu
from jax.experimental.pallas import tpu_sc as plsc

_NC = 2
_NS = 16
_NW = _NC * _NS
_L = 16


@functools.cache
def _gather_call(n_rows, vocab, batch, d):
    per_tile = -(-n_rows // _NW)
    half = batch // 2
    mesh = plsc.VectorSubcoreMesh(core_axis_name="c", subcore_axis_name="s")

    @functools.partial(
        pl.kernel,
        out_type=jax.ShapeDtypeStruct((n_rows, batch), jnp.float32),
        mesh=mesh,
        scratch_types=[
            pltpu.VMEM((vocab,), jnp.float32),
            pltpu.VMEM((batch,), jnp.int32),
            pltpu.VMEM((half,), jnp.float32),
            pltpu.SemaphoreType.DMA,
        ],
        compiler_params=pltpu.CompilerParams(needs_layout_passes=False),
    )
    def k(tab_hbm, idx_hbm, out_hbm, vbuf, ci, stage, sem):
        wid = lax.axis_index("s") * _NC + lax.axis_index("c")

        def pair_body(kk, carry):
            p = wid * per_tile + kk

            @pl.when(p < n_rows)
            def _():
                f = p // d
                j = lax.rem(p, d)
                pltpu.sync_copy(tab_hbm.at[f, j], vbuf)
                pltpu.sync_copy(idx_hbm.at[f], ci)
                for h in range(2):
                    def gather_body(i, c):
                        iv = ci[pl.ds(h * half + i * _L, _L)]
                        stage[pl.ds(i * _L, _L)] = plsc.load_gather(vbuf, [iv])
                        return c

                    lax.fori_loop(0, half // _L, gather_body, 0, unroll=8)
                    pltpu.sync_copy(stage, out_hbm.at[p, pl.ds(h * half, half)])

            return carry

        lax.fori_loop(0, per_tile, pair_body, 0)

    return k


def kernel(cat, tables):
    batch, n_fields = cat.shape
    _, vocab, d = tables.shape
    tab = tables.transpose(0, 2, 1)
    idx = cat.T
    out_t = _gather_call(n_fields * d, vocab, batch, d)(tab, idx)
    return out_t.T


# speedup vs baseline: 5.5104x; 1.0453x over previous
import functools

import jax
import jax.numpy as jnp
from jax import lax
from jax.experimental import pallas as pl
from jax.experimental.pallas import tpu as pltpu
from jax.experimental.pallas import tpu_sc as plsc

_NC = 2
_NS = 16
_NW = _NC * _NS
_L = 16


@functools.cache
def _gather_call(n_rows, vocab, batch, d):
    per_tile = -(-n_rows // _NW)
    n_vc = 4
    vc = vocab // n_vc
    n_q = 4
    q = batch // n_q
    mesh = plsc.VectorSubcoreMesh(core_axis_name="c", subcore_axis_name="s")

    @functools.partial(
        pl.kernel,
        out_type=jax.ShapeDtypeStruct((n_rows, batch), jnp.float32),
        mesh=mesh,
        scratch_types=[
            pltpu.VMEM((vocab,), jnp.float32),
            pltpu.VMEM((batch,), jnp.int32),
            pltpu.VMEM((2, q), jnp.float32),
            pltpu.SemaphoreType.DMA,
            pltpu.SemaphoreType.DMA,
        ],
        compiler_params=pltpu.CompilerParams(needs_layout_passes=False),
    )
    def k(tab_hbm, idx_hbm, out_hbm, vbuf, ci, stage, sem_in, sem_out):
        wid = lax.axis_index("s") * _NC + lax.axis_index("c")

        def pair_body(kk, carry):
            p = wid * per_tile + kk

            @pl.when(p < n_rows)
            def _():
                f = p // d
                j = lax.rem(p, d)
                in_descs = [
                    pltpu.async_copy(tab_hbm.at[f, j], vbuf, sem_in),
                    pltpu.async_copy(idx_hbm.at[f], ci, sem_in),
                ]
                for dd in in_descs:
                    dd.wait()
                out_descs = [None, None]
                for h in range(n_q):
                    if out_descs[h % 2] is not None:
                        out_descs[h % 2].wait()

                    def gather_body(i, c):
                        iv = ci[pl.ds(h * q + i * _L, _L)]
                        stage[h % 2, pl.ds(i * _L, _L)] = plsc.load_gather(vbuf, [iv])
                        return c

                    lax.fori_loop(0, q // _L, gather_body, 0, unroll=8)
                    out_descs[h % 2] = pltpu.async_copy(
                        stage.at[h % 2], out_hbm.at[p, pl.ds(h * q, q)], sem_out
                    )
                for dd in out_descs:
                    dd.wait()

            return carry

        lax.fori_loop(0, per_tile, pair_body, 0)

    return k


def kernel(cat, tables):
    batch, n_fields = cat.shape
    _, vocab, d = tables.shape
    tab = tables.transpose(0, 2, 1)
    idx = cat.T
    out_t = _gather_call(n_fields * d, vocab, batch, d)(tab, idx)
    return out_t.T


# speedup vs baseline: 5.5140x; 1.0007x over previous
import functools

import jax
import jax.numpy as jnp
from jax import lax
from jax.experimental import pallas as pl
from jax.experimental.pallas import tpu as pltpu
from jax.experimental.pallas import tpu_sc as plsc

_NC = 2
_NS = 16
_NW = _NC * _NS
_L = 16


@functools.cache
def _gather_call(n_rows, vocab, batch, d):
    per_tile = -(-n_rows // _NW)
    n_q = 4
    q = batch // n_q
    mesh = plsc.VectorSubcoreMesh(core_axis_name="c", subcore_axis_name="s")

    @functools.partial(
        pl.kernel,
        out_type=jax.ShapeDtypeStruct((n_rows, batch), jnp.float32),
        mesh=mesh,
        scratch_types=[
            pltpu.VMEM((vocab,), jnp.float32),
            pltpu.VMEM((batch,), jnp.int32),
            pltpu.VMEM((2, q), jnp.float32),
            pltpu.SemaphoreType.DMA,
            pltpu.SemaphoreType.DMA,
        ],
        compiler_params=pltpu.CompilerParams(needs_layout_passes=False),
    )
    def k(tab_hbm, idx_hbm, out_hbm, vbuf, ci, stage, sem_in, sem_out):
        wid = lax.axis_index("s") * _NC + lax.axis_index("c")

        def pair_body(kk, carry):
            p = wid * per_tile + kk

            @pl.when(p < n_rows)
            def _():
                f = p // d
                j = lax.rem(p, d)
                in_descs = [
                    pltpu.async_copy(tab_hbm.at[f, j], vbuf, sem_in),
                    pltpu.async_copy(idx_hbm.at[f], ci, sem_in),
                ]
                for dd in in_descs:
                    dd.wait()
                out_descs = [None, None]
                for h in range(n_q):
                    if out_descs[h % 2] is not None:
                        out_descs[h % 2].wait()

                    def gather_body(i, c):
                        iv = ci[pl.ds(h * q + i * _L, _L)]
                        stage[h % 2, pl.ds(i * _L, _L)] = plsc.load_gather(vbuf, [iv])
                        return c

                    lax.fori_loop(0, q // _L, gather_body, 0, unroll=8)
                    out_descs[h % 2] = pltpu.async_copy(
                        stage.at[h % 2], out_hbm.at[p, pl.ds(h * q, q)], sem_out
                    )
                for dd in out_descs:
                    dd.wait()

            return carry

        lax.fori_loop(0, per_tile, pair_body, 0)

    return k


def kernel(cat, tables):
    batch, n_fields = cat.shape
    _, vocab, d = tables.shape
    tab = tables.transpose(0, 2, 1)
    idx = cat.T
    out_t = _gather_call(n_fields * d, vocab, batch, d)(tab, idx)
    return out_t.T


# speedup vs baseline: 5.7424x; 1.0414x over previous
import functools

import jax
import jax.numpy as jnp
from jax import lax
from jax.experimental import pallas as pl
from jax.experimental.pallas import tpu as pltpu
from jax.experimental.pallas import tpu_sc as plsc

_NC = 2
_NS = 16
_NW = _NC * _NS
_L = 16


@functools.cache
def _gather_call(n_rows, vocab, batch, d):
    per_tile = -(-n_rows // _NW)
    n_q = 4
    q = batch // n_q
    mesh = plsc.VectorSubcoreMesh(core_axis_name="c", subcore_axis_name="s")

    @functools.partial(
        pl.kernel,
        out_type=jax.ShapeDtypeStruct((n_rows, batch), jnp.float32),
        mesh=mesh,
        scratch_types=[
            pltpu.VMEM((vocab,), jnp.float32),
            pltpu.VMEM((batch,), jnp.int32),
            pltpu.VMEM((2, q), jnp.float32),
            pltpu.SemaphoreType.DMA,
            pltpu.SemaphoreType.DMA,
        ],
        compiler_params=pltpu.CompilerParams(needs_layout_passes=False),
    )
    def k(tab_hbm, idx_hbm, out_hbm, vbuf, ci, stage, sem_in, sem_out):
        wid = lax.axis_index("s") * _NC + lax.axis_index("c")

        def pair_body(kk, f_prev):
            p = wid * per_tile + kk
            f = jnp.minimum(p, n_rows - 1) // d

            @pl.when(p < n_rows)
            def _():
                j = lax.rem(p, d)
                vd = pltpu.async_copy(tab_hbm.at[f, j], vbuf, sem_in)

                @pl.when(f != f_prev)
                def _():
                    pltpu.sync_copy(idx_hbm.at[f], ci)

                vd.wait()
                out_descs = [None, None]
                for h in range(n_q):
                    if out_descs[h % 2] is not None:
                        out_descs[h % 2].wait()

                    def gather_body(i, c):
                        iv = ci[pl.ds(h * q + i * _L, _L)]
                        stage[h % 2, pl.ds(i * _L, _L)] = plsc.load_gather(vbuf, [iv])
                        return c

                    lax.fori_loop(0, q // _L, gather_body, 0, unroll=8)
                    out_descs[h % 2] = pltpu.async_copy(
                        stage.at[h % 2], out_hbm.at[p, pl.ds(h * q, q)], sem_out
                    )
                for dd in out_descs:
                    dd.wait()

            return f

        lax.fori_loop(0, per_tile, pair_body, jnp.int32(-1))

    return k


def kernel(cat, tables):
    batch, n_fields = cat.shape
    _, vocab, d = tables.shape
    tab = tables.transpose(0, 2, 1)
    idx = cat.T
    out_t = _gather_call(n_fields * d, vocab, batch, d)(tab, idx)
    return out_t.T


# speedup vs baseline: 5.7636x; 1.0037x over previous
import functools

import jax
import jax.numpy as jnp
from jax import lax
from jax.experimental import pallas as pl
from jax.experimental.pallas import tpu as pltpu
from jax.experimental.pallas import tpu_sc as plsc

_NC = 2
_NS = 16
_NW = _NC * _NS
_L = 16


@functools.cache
def _gather_call(n_rows, vocab, batch, d):
    per_tile = -(-n_rows // _NW)
    n_q = 4
    q = batch // n_q
    mesh = plsc.VectorSubcoreMesh(core_axis_name="c", subcore_axis_name="s")

    @functools.partial(
        pl.kernel,
        out_type=jax.ShapeDtypeStruct((n_rows, batch), jnp.float32),
        mesh=mesh,
        scratch_types=[
            pltpu.VMEM((vocab,), jnp.float32),
            pltpu.VMEM((batch,), jnp.int32),
            pltpu.VMEM((2, q), jnp.float32),
            pltpu.SemaphoreType.DMA,
            pltpu.SemaphoreType.DMA,
        ],
        compiler_params=pltpu.CompilerParams(needs_layout_passes=False),
    )
    def k(tab_hbm, idx_hbm, out_hbm, vbuf, ci, stage, sem_in, sem_out):
        wid = lax.axis_index("s") * _NC + lax.axis_index("c")

        def pair_body(kk, f_prev):
            p = wid * per_tile + kk
            f = jnp.minimum(p, n_rows - 1) // d

            @pl.when(p < n_rows)
            def _():
                j = lax.rem(p, d)
                vd = pltpu.async_copy(tab_hbm.at[f, j], vbuf, sem_in)

                @pl.when(f != f_prev)
                def _():
                    pltpu.sync_copy(idx_hbm.at[f], ci)

                vd.wait()
                out_descs = [None, None]
                for h in range(n_q):
                    if out_descs[h % 2] is not None:
                        out_descs[h % 2].wait()

                    def gather_body(i, c):
                        iv = ci[pl.ds(h * q + i * _L, _L)]
                        stage[h % 2, pl.ds(i * _L, _L)] = plsc.load_gather(vbuf, [iv])
                        return c

                    lax.fori_loop(0, q // _L, gather_body, 0, unroll=16)
                    out_descs[h % 2] = pltpu.async_copy(
                        stage.at[h % 2], out_hbm.at[p, pl.ds(h * q, q)], sem_out
                    )
                for dd in out_descs:
                    dd.wait()

            return f

        lax.fori_loop(0, per_tile, pair_body, jnp.int32(-1))

    return k


def kernel(cat, tables):
    batch, n_fields = cat.shape
    _, vocab, d = tables.shape
    tab = tables.transpose(0, 2, 1)
    idx = cat.T
    out_t = _gather_call(n_fields * d, vocab, batch, d)(tab, idx)
    return out_t.T
